# Initial kernel scaffold; baseline (speedup 1.0000x reference)
import jax
import jax.numpy as jnp
from jax.experimental import pallas as pl


def kernel(x, y, vecs_use, W1, b1, W2, b2, Wv1, bv1, Wv2, bv2, candidate_number):
    raise NotImplementedError("write your pallas kernel here")


# speedup vs baseline: 10.8611x; 10.8611x over previous
import jax
import jax.numpy as jnp
from jax.experimental import pallas as pl

_BBLK = 256


def _retrieve_body(h_ref, h2_ref, nT_ref, n2_ref, self_ref, out_ref):
    Bblk = h_ref.shape[0]
    C = nT_ref.shape[1]
    h = h_ref[...]
    hn = jax.lax.dot_general(h, nT_ref[...], (((1,), (0,)), ((), ())),
                             preferred_element_type=jnp.float32)
    d2 = (h2_ref[...] + n2_ref[...]) - 2.0 * hn
    dist = jnp.sqrt(jnp.maximum(d2, 0.0) + 1e-12)
    score = jnp.exp(-dist * 0.02)

    iota = jax.lax.broadcasted_iota(jnp.int32, (Bblk, C), 1)
    A = score
    M = jnp.zeros((Bblk, C), jnp.float32)
    for _ in range(10):
        m = jnp.max(A, axis=1, keepdims=True)
        fi = jnp.min(jnp.where(A == m, iota, C), axis=1, keepdims=True)
        sel = iota == fi
        M = jnp.where(sel, 1.0, M)
        A = jnp.where(sel, -1.0, A)

    w = M * jnp.exp(score - 1.0)
    Z = jnp.sum(w, axis=1, keepdims=True) + 1.0
    outp = jax.lax.dot_general(w, nT_ref[...], (((1,), (1,)), ((), ())),
                               preferred_element_type=jnp.float32)
    out_ref[...] = (outp + self_ref[...]) / Z


def kernel(x, y, vecs_use, W1, b1, W2, b2, Wv1, bv1, Wv2, bv2, candidate_number):
    S, U = x.shape
    V, E = vecs_use.shape
    H = W2.shape[1]
    C = candidate_number.shape[0]
    B = S * U
    xv = x.reshape(-1)

    x_emb = jnp.take(vecs_use, xv, axis=0).reshape(S, U, E)

    def shift(n):
        return jnp.concatenate([x_emb[0:n], x_emb[0:S - n]], axis=0).reshape(-1, E)

    concat = jnp.concatenate(
        [shift(4), shift(3), shift(2), shift(1), x_emb.reshape(-1, E)], axis=-1)
    h = jnp.dot(jax.nn.relu(jnp.dot(concat, W1) + b1), W2) + b2
    x_dy = jnp.dot(jax.nn.relu(jnp.dot(vecs_use, Wv1) + bv1), Wv2) + bv2
    neigh = jnp.take(x_dy, candidate_number, axis=0)
    neighT = neigh.T
    n2 = jnp.sum(neigh * neigh, axis=-1)[None, :]
    h2 = jnp.sum(h * h, axis=-1, keepdims=True)
    self_emb = jnp.take(x_dy, xv, axis=0)

    grid = (B // _BBLK,)
    out = pl.pallas_call(
        _retrieve_body,
        grid=grid,
        in_specs=[
            pl.BlockSpec((_BBLK, H), lambda i: (i, 0)),
            pl.BlockSpec((_BBLK, 1), lambda i: (i, 0)),
            pl.BlockSpec((H, C), lambda i: (0, 0)),
            pl.BlockSpec((1, C), lambda i: (0, 0)),
            pl.BlockSpec((_BBLK, H), lambda i: (i, 0)),
        ],
        out_specs=pl.BlockSpec((_BBLK, H), lambda i: (i, 0)),
        out_shape=jax.ShapeDtypeStruct((B, H), jnp.float32),
    )(h, h2, neighT, n2, self_emb)
    return out


# speedup vs baseline: 13.8595x; 1.2761x over previous
import jax
import jax.numpy as jnp
from jax.experimental import pallas as pl

_BBLK = 256


def _retrieve_body(h_ref, h2_ref, nT_ref, n2_ref, self_ref, out_ref):
    Bblk = h_ref.shape[0]
    C = nT_ref.shape[1]
    h = h_ref[...]
    hn = jax.lax.dot_general(h, nT_ref[...], (((1,), (0,)), ((), ())),
                             preferred_element_type=jnp.float32)
    d2 = (h2_ref[...] + n2_ref[...]) - 2.0 * hn
    dist = jnp.sqrt(jnp.maximum(d2, 0.0) + 1e-12)
    score = jnp.exp(-dist * 0.02)

    iota = jax.lax.broadcasted_iota(jnp.int32, (Bblk, C), 1)
    lo = jnp.min(score, axis=1, keepdims=True)
    hi = jnp.max(score, axis=1, keepdims=True)
    scale = (float(2 ** 20 - 2)) / jnp.maximum(hi - lo, 1e-20)
    qv = ((score - lo) * scale).astype(jnp.int32)
    key0 = qv * C + (C - 1 - iota)
    K = key0
    for _ in range(9):
        m = jnp.max(K, axis=1, keepdims=True)
        K = jnp.where(K == m, -1, K)
    k10 = jnp.max(K, axis=1, keepdims=True)
    M = key0 >= k10

    w = jnp.where(M, jnp.exp(score - 1.0), 0.0)
    Z = jnp.sum(w, axis=1, keepdims=True) + 1.0
    outp = jax.lax.dot_general(w, nT_ref[...], (((1,), (1,)), ((), ())),
                               preferred_element_type=jnp.float32)
    out_ref[...] = (outp + self_ref[...]) / Z


def kernel(x, y, vecs_use, W1, b1, W2, b2, Wv1, bv1, Wv2, bv2, candidate_number):
    S, U = x.shape
    V, E = vecs_use.shape
    H = W2.shape[1]
    C = candidate_number.shape[0]
    B = S * U
    xv = x.reshape(-1)

    x_emb = jnp.take(vecs_use, xv, axis=0).reshape(S, U, E)

    def shift(n):
        return jnp.concatenate([x_emb[0:n], x_emb[0:S - n]], axis=0).reshape(-1, E)

    concat = jnp.concatenate(
        [shift(4), shift(3), shift(2), shift(1), x_emb.reshape(-1, E)], axis=-1)
    h = jnp.dot(jax.nn.relu(jnp.dot(concat, W1) + b1), W2) + b2
    x_dy = jnp.dot(jax.nn.relu(jnp.dot(vecs_use, Wv1) + bv1), Wv2) + bv2
    neigh = jnp.take(x_dy, candidate_number, axis=0)
    neighT = neigh.T
    n2 = jnp.sum(neigh * neigh, axis=-1)[None, :]
    h2 = jnp.sum(h * h, axis=-1, keepdims=True)
    self_emb = jnp.take(x_dy, xv, axis=0)

    grid = (B // _BBLK,)
    out = pl.pallas_call(
        _retrieve_body,
        grid=grid,
        in_specs=[
            pl.BlockSpec((_BBLK, H), lambda i: (i, 0)),
            pl.BlockSpec((_BBLK, 1), lambda i: (i, 0)),
            pl.BlockSpec((H, C), lambda i: (0, 0)),
            pl.BlockSpec((1, C), lambda i: (0, 0)),
            pl.BlockSpec((_BBLK, H), lambda i: (i, 0)),
        ],
        out_specs=pl.BlockSpec((_BBLK, H), lambda i: (i, 0)),
        out_shape=jax.ShapeDtypeStruct((B, H), jnp.float32),
    )(h, h2, neighT, n2, self_emb)
    return out


# speedup vs baseline: 14.0673x; 1.0150x over previous
import functools

import jax
import jax.numpy as jnp
from jax import lax
from jax.experimental import pallas as pl
from jax.experimental.pallas import tpu as pltpu
from jax.experimental.pallas import tpu_sc as plsc

_BBLK = 256
_VBLK = 4000


def _sc_gather(table, idx, out_dtype=jnp.float32):
    V, D = table.shape
    B = idx.shape[0]
    info = plsc.get_sparse_core_info()
    NC, NS = info.num_cores, info.num_subcores
    NW = NC * NS
    assert B % (8 * NW) == 0, (B, NW)
    b_per_w = B // NW
    mesh = plsc.VectorSubcoreMesh(core_axis_name="c", subcore_axis_name="s")

    ch = 128 if b_per_w % 128 == 0 else b_per_w
    assert b_per_w % ch == 0 and ch <= 128, (b_per_w, ch)
    nch = b_per_w // ch

    @functools.partial(
        pl.kernel, mesh=mesh,
        compiler_params=pltpu.CompilerParams(use_tc_tiling_on_sc=False),
        out_type=jax.ShapeDtypeStruct((B, D), out_dtype),
        scratch_types=[
            pltpu.VMEM((b_per_w,), jnp.int32),
            pltpu.VMEM((b_per_w, D), out_dtype),
            pltpu.SemaphoreType.DMA,
        ],
    )
    def k(table_hbm, idx_hbm, out_hbm, idx_v, rows_v, sem):
        wid = lax.axis_index("s") * NC + lax.axis_index("c")
        base = wid * b_per_w
        pltpu.sync_copy(idx_hbm.at[pl.ds(base, b_per_w)], idx_v)
        copies = [
            pltpu.async_copy(table_hbm.at[idx_v.at[pl.ds(j * ch, ch)]],
                             rows_v.at[pl.ds(j * ch, ch)], sem)
            for j in range(nch)
        ]
        for cpy in copies:
            cpy.wait()
        pltpu.sync_copy(rows_v, out_hbm.at[pl.ds(base, b_per_w)])

    return k(table, idx)


def _xdy_body(v_ref, Wv1_ref, bv1_ref, Wv2_ref, bv2_ref, out_ref):
    v = v_ref[...]
    t = jnp.maximum(
        jax.lax.dot_general(v, Wv1_ref[...], (((1,), (0,)), ((), ())),
                            preferred_element_type=jnp.float32) + bv1_ref[...], 0.0)
    out_ref[...] = jax.lax.dot_general(
        t, Wv2_ref[...], (((1,), (0,)), ((), ())),
        preferred_element_type=jnp.float32) + bv2_ref[...]


def _retrieve_body(e4_ref, e3_ref, e2_ref, e1_ref, e0_ref, W1_ref, b1_ref,
                   W2_ref, b2_ref, nT_ref, self_ref, out_ref):
    Bblk = e0_ref.shape[0]
    C = nT_ref.shape[1]
    concat = jnp.concatenate(
        [e4_ref[...], e3_ref[...], e2_ref[...], e1_ref[...], e0_ref[...]], axis=1)
    t = jnp.maximum(
        jax.lax.dot_general(concat, W1_ref[...], (((1,), (0,)), ((), ())),
                            preferred_element_type=jnp.float32) + b1_ref[...], 0.0)
    h = jax.lax.dot_general(t, W2_ref[...], (((1,), (0,)), ((), ())),
                            preferred_element_type=jnp.float32) + b2_ref[...]
    nT = nT_ref[...]
    h2 = jnp.sum(h * h, axis=1, keepdims=True)
    n2 = jnp.sum(nT * nT, axis=0, keepdims=True)
    hn = jax.lax.dot_general(h, nT, (((1,), (0,)), ((), ())),
                             preferred_element_type=jnp.float32)
    d2 = (h2 + n2) - 2.0 * hn
    dist = jnp.sqrt(jnp.maximum(d2, 0.0) + 1e-12)
    score = jnp.exp(-dist * 0.02)

    iota = jax.lax.broadcasted_iota(jnp.int32, (Bblk, C), 1)
    lo = jnp.min(score, axis=1, keepdims=True)
    hi = jnp.max(score, axis=1, keepdims=True)
    scale = (float(2 ** 20 - 2)) / jnp.maximum(hi - lo, 1e-20)
    qv = ((score - lo) * scale).astype(jnp.int32)
    key0 = qv * C + (C - 1 - iota)
    K = key0
    for _ in range(9):
        m = jnp.max(K, axis=1, keepdims=True)
        K = jnp.where(K == m, -1, K)
    k10 = jnp.max(K, axis=1, keepdims=True)
    M = key0 >= k10

    w = jnp.where(M, jnp.exp(score - 1.0), 0.0)
    Z = jnp.sum(w, axis=1, keepdims=True) + 1.0
    outp = jax.lax.dot_general(w, nT, (((1,), (1,)), ((), ())),
                               preferred_element_type=jnp.float32)
    out_ref[...] = (outp + self_ref[...]) / Z


def kernel(x, y, vecs_use, W1, b1, W2, b2, Wv1, bv1, Wv2, bv2, candidate_number):
    S, U = x.shape
    V, E = vecs_use.shape
    H = W2.shape[1]
    C = candidate_number.shape[0]
    B = S * U
    HP = 16
    xv = x.reshape(-1)

    EP = 32
    vecs_p = jnp.pad(vecs_use, ((0, 0), (0, EP - E)))
    x_emb = _sc_gather(vecs_p, xv)[:, :E]

    Wv2p = jnp.pad(Wv2, ((0, 0), (0, HP - H)))
    bv2p = jnp.pad(bv2, (0, HP - H))[None, :]
    x_dy = pl.pallas_call(
        _xdy_body,
        grid=(V // _VBLK,),
        in_specs=[
            pl.BlockSpec((_VBLK, E), lambda i: (i, 0)),
            pl.BlockSpec((E, E), lambda i: (0, 0)),
            pl.BlockSpec((1, E), lambda i: (0, 0)),
            pl.BlockSpec((E, HP), lambda i: (0, 0)),
            pl.BlockSpec((1, HP), lambda i: (0, 0)),
        ],
        out_specs=pl.BlockSpec((_VBLK, HP), lambda i: (i, 0)),
        out_shape=jax.ShapeDtypeStruct((V, HP), jnp.float32),
    )(vecs_use, Wv1, bv1[None, :], Wv2p, bv2p)

    neigh = _sc_gather(x_dy, candidate_number.astype(jnp.int32))
    self_emb = _sc_gather(x_dy, xv)
    neighT = neigh.T

    W2p = jnp.pad(W2, ((0, 0), (0, HP - H)))
    b2p = jnp.pad(b2, (0, HP - H))[None, :]

    nblk = U // _BBLK

    def emb_spec(n):
        def imap(i):
            s = i // nblk
            return (jnp.where(s < n, s, s - n) * nblk + i % nblk, 0)
        return pl.BlockSpec((_BBLK, E), imap)

    out = pl.pallas_call(
        _retrieve_body,
        grid=(B // _BBLK,),
        in_specs=[
            emb_spec(4), emb_spec(3), emb_spec(2), emb_spec(1), emb_spec(0),
            pl.BlockSpec((5 * E, E), lambda i: (0, 0)),
            pl.BlockSpec((1, E), lambda i: (0, 0)),
            pl.BlockSpec((E, HP), lambda i: (0, 0)),
            pl.BlockSpec((1, HP), lambda i: (0, 0)),
            pl.BlockSpec((HP, C), lambda i: (0, 0)),
            pl.BlockSpec((_BBLK, HP), lambda i: (i, 0)),
        ],
        out_specs=pl.BlockSpec((_BBLK, HP), lambda i: (i, 0)),
        out_shape=jax.ShapeDtypeStruct((B, HP), jnp.float32),
    )(x_emb, x_emb, x_emb, x_emb, x_emb, W1, b1[None, :], W2p, b2p,
      neighT, self_emb)
    return out[:, :H]
